# baseline (speedup 1.0000x reference)
import functools

import jax
import jax.numpy as jnp
from jax import lax
from jax.experimental import pallas as pl
from jax.experimental.pallas import tpu as pltpu
from jax.experimental.pallas import tpu_sc as plsc

B = 4096
L = 200
DIM = 32
N_LABELS = 10
VOCAB = 1000000
DP = 16

NC = 2
NS = 16
NW = NC * NS
ROWS_PER_W = B // NW
CHUNK_ROWS = 16
CHUNK_TOKS = CHUNK_ROWS * L
N_CHUNKS = ROWS_PER_W // CHUNK_ROWS
GATHER_N = 128
N_G = CHUNK_TOKS // GATHER_N

PACK = 128 // DP
VBLK = 32768
SUB = VBLK // PACK
NBLK = -(-VOCAB // VBLK)
TW_ROWS = NBLK * VBLK // PACK


def _tw_body(t_ref, wt_ref, o_ref):
    parts = [jnp.dot(wt_ref[...], t_ref[:, m * SUB:(m + 1) * SUB],
                     preferred_element_type=jnp.float32)
             for m in range(PACK)]
    p8 = jnp.concatenate(parts, axis=0).astype(jnp.bfloat16)
    o_ref[...] = jnp.transpose(p8).astype(jnp.float32)


def _tw(tabT, WpT):
    return pl.pallas_call(
        _tw_body,
        grid=(NBLK,),
        in_specs=[
            pl.BlockSpec((DIM, VBLK), lambda j: (0, j)),
            pl.BlockSpec((DP, DIM), lambda j: (0, 0)),
        ],
        out_specs=pl.BlockSpec((SUB, 128), lambda j: (j, 0)),
        out_shape=jax.ShapeDtypeStruct((TW_ROWS, 128), jnp.float32),
    )(tabT, WpT)


def _stage_idx(xt_hbm, wid, c, stage_v, idx_v):
    row_base = wid * ROWS_PER_W + c * CHUNK_ROWS
    pltpu.sync_copy(xt_hbm.at[:, pl.ds(row_base, CHUNK_ROWS)], stage_v)

    def remap_body(g, _):
        for u in range(GATHER_N // CHUNK_ROWS):
            v = stage_v[g * (GATHER_N // CHUNK_ROWS) + u, :]
            c_lo = v & (VBLK - 1)
            m = c_lo >> 12
            k = c_lo & (SUB - 1)
            idx_v[g, pl.ds(u * CHUNK_ROWS, CHUNK_ROWS)] = \
                v - c_lo + (k << 3) + m
        return 0

    lax.fori_loop(0, N_G, remap_body, 0)


def _fire_gathers(tw_hbm, idx_v, rows_v, sem):
    copies = []
    for j in range(N_G):
        copies.append(pltpu.async_copy(
            tw_hbm.at[idx_v.at[j]],
            rows_v.at[pl.ds(j * GATHER_N, GATHER_N)],
            sem))
    return copies


def _chunk_reduce(out_hbm, wid, c, rows_v, acc_v, bias_v):
    row_base = wid * ROWS_PER_W + c * CHUNK_ROWS

    def row_body(r, _):
        def tok_body(t, a):
            return a + rows_v[t * CHUNK_ROWS + r, :]
        a = lax.fori_loop(0, L, tok_body, bias_v[...], unroll=8)
        acc_v[r, :] = a
        return 0

    lax.fori_loop(0, CHUNK_ROWS, row_body, 0)
    pltpu.sync_copy(acc_v, out_hbm.at[pl.ds(row_base, CHUNK_ROWS)])


def _pool_body(xt_hbm, tw_hbm, bias_hbm, out_hbm,
               stage_v, idx0_v, idx1_v, rows0_v, rows1_v, acc_v, bias_v,
               sem0, sem1):
    wid = lax.axis_index("s") * NC + lax.axis_index("c")
    pltpu.sync_copy(bias_hbm, bias_v)

    idx_bufs = (idx0_v, idx1_v)
    rows_bufs = (rows0_v, rows1_v)
    sems = (sem0, sem1)

    _stage_idx(xt_hbm, wid, 0, stage_v, idx_bufs[0])
    pending = _fire_gathers(tw_hbm, idx_bufs[0], rows_bufs[0], sems[0])
    nxt = []
    for c in range(N_CHUNKS):
        if c + 1 < N_CHUNKS:
            _stage_idx(xt_hbm, wid, c + 1, stage_v, idx_bufs[(c + 1) % 2])
            nxt = _fire_gathers(tw_hbm, idx_bufs[(c + 1) % 2],
                                rows_bufs[(c + 1) % 2], sems[(c + 1) % 2])
        for cp in pending:
            cp.wait()
        pending = nxt
        _chunk_reduce(out_hbm, wid, c, rows_bufs[c % 2], acc_v, bias_v)


_pool = functools.partial(
    pl.kernel,
    mesh=plsc.VectorSubcoreMesh(core_axis_name="c", subcore_axis_name="s"),
    compiler_params=pltpu.CompilerParams(use_tc_tiling_on_sc=False),
    out_type=jax.ShapeDtypeStruct((B, DP), jnp.float32),
    scratch_types=[
        pltpu.VMEM((L, CHUNK_ROWS), jnp.int32),
        pltpu.VMEM((N_G, GATHER_N), jnp.int32),
        pltpu.VMEM((N_G, GATHER_N), jnp.int32),
        pltpu.VMEM((CHUNK_TOKS, DP), jnp.float32),
        pltpu.VMEM((CHUNK_TOKS, DP), jnp.float32),
        pltpu.VMEM((CHUNK_ROWS, DP), jnp.float32),
        pltpu.VMEM((DP,), jnp.float32),
        pltpu.SemaphoreType.DMA,
        pltpu.SemaphoreType.DMA,
    ],
)(_pool_body)


def kernel(x, seg, mask, tok_embed, W, b):
    tabT = tok_embed.T
    WpT = jnp.pad(W, ((0, 0), (0, DP - N_LABELS))).T
    bp = jnp.pad(b, (0, DP - N_LABELS))
    tw128 = _tw(tabT, WpT)
    tw = tw128.reshape(TW_ROWS * PACK, DP)
    xt = x.T.astype(jnp.int32)
    pooled = _pool(xt, tw, bp)
    return pooled[:, :N_LABELS]


# speedup vs baseline: 1.0200x; 1.0200x over previous
import functools

import jax
import jax.numpy as jnp
from jax import lax
from jax.experimental import pallas as pl
from jax.experimental.pallas import tpu as pltpu
from jax.experimental.pallas import tpu_sc as plsc

B = 4096
L = 200
DIM = 32
N_LABELS = 10
VOCAB = 1000000
DP = 16

NC = 2
NS = 16
NW = NC * NS
ROWS_PER_W = B // NW
CHUNK_ROWS = 16
CHUNK_TOKS = CHUNK_ROWS * L
N_CHUNKS = ROWS_PER_W // CHUNK_ROWS
GATHER_N = 128
N_G = CHUNK_TOKS // GATHER_N

PACK = 128 // DP
VBLK = 65536
SUB = VBLK // PACK
NBLK = -(-VOCAB // VBLK)
TW_ROWS = NBLK * VBLK // PACK


def _tw_body(t_ref, wt_ref, o_ref):
    parts = [jnp.dot(wt_ref[...], t_ref[:, m * SUB:(m + 1) * SUB],
                     preferred_element_type=jnp.float32)
             for m in range(PACK)]
    p8 = jnp.concatenate(parts, axis=0).astype(jnp.bfloat16)
    o_ref[...] = jnp.transpose(p8).astype(jnp.float32)


def _tw(tabT, WpT):
    return pl.pallas_call(
        _tw_body,
        grid=(NBLK,),
        in_specs=[
            pl.BlockSpec((DIM, VBLK), lambda j: (0, j)),
            pl.BlockSpec((DP, DIM), lambda j: (0, 0)),
        ],
        out_specs=pl.BlockSpec((SUB, 128), lambda j: (j, 0)),
        out_shape=jax.ShapeDtypeStruct((TW_ROWS, 128), jnp.float32),
    )(tabT, WpT)


def _stage_idx(xt_hbm, wid, c, stage_v, idx_v):
    row_base = wid * ROWS_PER_W + c * CHUNK_ROWS
    pltpu.sync_copy(xt_hbm.at[:, pl.ds(row_base, CHUNK_ROWS)], stage_v)

    def remap_body(g, _):
        for u in range(GATHER_N // CHUNK_ROWS):
            v = stage_v[g * (GATHER_N // CHUNK_ROWS) + u, :]
            c_lo = v & (VBLK - 1)
            m = c_lo >> 13
            k = c_lo & (SUB - 1)
            idx_v[g, pl.ds(u * CHUNK_ROWS, CHUNK_ROWS)] = \
                v - c_lo + (k << 3) + m
        return 0

    lax.fori_loop(0, N_G, remap_body, 0)


def _fire_gathers(tw_hbm, idx_v, rows_v, sem):
    copies = []
    for j in range(N_G):
        copies.append(pltpu.async_copy(
            tw_hbm.at[idx_v.at[j]],
            rows_v.at[pl.ds(j * GATHER_N, GATHER_N)],
            sem))
    return copies


def _chunk_reduce(out_hbm, wid, c, rows_v, acc_v, bias_v):
    row_base = wid * ROWS_PER_W + c * CHUNK_ROWS

    def row_body(r, _):
        def tok_body(t, a):
            return a + rows_v[t * CHUNK_ROWS + r, :]
        a = lax.fori_loop(0, L, tok_body, bias_v[...], unroll=8)
        acc_v[r, :] = a
        return 0

    lax.fori_loop(0, CHUNK_ROWS, row_body, 0)
    pltpu.sync_copy(acc_v, out_hbm.at[pl.ds(row_base, CHUNK_ROWS)])


def _pool_body(xt_hbm, tw_hbm, bias_hbm, out_hbm,
               stage_v, idx0_v, idx1_v, rows0_v, rows1_v, acc_v, bias_v,
               sem0, sem1):
    wid = lax.axis_index("s") * NC + lax.axis_index("c")
    pltpu.sync_copy(bias_hbm, bias_v)

    idx_bufs = (idx0_v, idx1_v)
    rows_bufs = (rows0_v, rows1_v)
    sems = (sem0, sem1)

    _stage_idx(xt_hbm, wid, 0, stage_v, idx_bufs[0])
    pending = _fire_gathers(tw_hbm, idx_bufs[0], rows_bufs[0], sems[0])
    nxt = []
    for c in range(N_CHUNKS):
        if c + 1 < N_CHUNKS:
            _stage_idx(xt_hbm, wid, c + 1, stage_v, idx_bufs[(c + 1) % 2])
            nxt = _fire_gathers(tw_hbm, idx_bufs[(c + 1) % 2],
                                rows_bufs[(c + 1) % 2], sems[(c + 1) % 2])
        for cp in pending:
            cp.wait()
        pending = nxt
        _chunk_reduce(out_hbm, wid, c, rows_bufs[c % 2], acc_v, bias_v)


_pool = functools.partial(
    pl.kernel,
    mesh=plsc.VectorSubcoreMesh(core_axis_name="c", subcore_axis_name="s"),
    compiler_params=pltpu.CompilerParams(use_tc_tiling_on_sc=False),
    out_type=jax.ShapeDtypeStruct((B, DP), jnp.float32),
    scratch_types=[
        pltpu.VMEM((L, CHUNK_ROWS), jnp.int32),
        pltpu.VMEM((N_G, GATHER_N), jnp.int32),
        pltpu.VMEM((N_G, GATHER_N), jnp.int32),
        pltpu.VMEM((CHUNK_TOKS, DP), jnp.float32),
        pltpu.VMEM((CHUNK_TOKS, DP), jnp.float32),
        pltpu.VMEM((CHUNK_ROWS, DP), jnp.float32),
        pltpu.VMEM((DP,), jnp.float32),
        pltpu.SemaphoreType.DMA,
        pltpu.SemaphoreType.DMA,
    ],
)(_pool_body)


def kernel(x, seg, mask, tok_embed, W, b):
    tabT = tok_embed.T
    WpT = jnp.pad(W, ((0, 0), (0, DP - N_LABELS))).T
    bp = jnp.pad(b, (0, DP - N_LABELS))
    tw128 = _tw(tabT, WpT)
    tw = tw128.reshape(TW_ROWS * PACK, DP)
    xt = x.T.astype(jnp.int32)
    pooled = _pool(xt, tw, bp)
    return pooled[:, :N_LABELS]
